# baseline (speedup 1.0000x reference)
import functools

import jax
import jax.numpy as jnp
from jax import lax
from jax.experimental import pallas as pl
from jax.experimental.pallas import tpu as pltpu
from jax.experimental.pallas import tpu_sc as plsc

_B, _L = 16384, 200
_NC, _NS = 2, 16
_NW = _NC * _NS
_ROWS_W = _B // _NW
_CHUNK_R = 64
_NCHUNKS = _ROWS_W // _CHUNK_R
_TBL_PAD = 40
_LANES = 16
_COL_OFFS = tuple(range(0, 192, 16)) + (184,)

_mesh = plsc.VectorSubcoreMesh(core_axis_name="c", subcore_axis_name="s")


@functools.partial(
    pl.kernel,
    mesh=_mesh,
    out_type=jax.ShapeDtypeStruct((_B, _L), jnp.float32),
    scratch_types=[
        pltpu.VMEM((_TBL_PAD,), jnp.float32),
        pltpu.VMEM((2, _CHUNK_R, _L), jnp.int32),
        pltpu.VMEM((2, _CHUNK_R, _L), jnp.float32),
        pltpu.SemaphoreType.DMA,
        pltpu.SemaphoreType.DMA,
        pltpu.SemaphoreType.DMA,
        pltpu.SemaphoreType.DMA,
    ],
    compiler_params=pltpu.CompilerParams(
        needs_layout_passes=False, use_tc_tiling_on_sc=True),
)
def _lut_gather(idx_hbm, tbl_hbm, out_hbm, tbl_v, idx_v, out_v,
                sin0, sin1, sout0, sout1):
    wid = lax.axis_index("s") * _NC + lax.axis_index("c")
    base = wid * _ROWS_W
    sins = (sin0, sin1)
    souts = (sout0, sout1)
    pltpu.sync_copy(tbl_hbm, tbl_v)

    def start_in(ci):
        off = base + ci * _CHUNK_R
        return pltpu.async_copy(idx_hbm.at[pl.ds(off, _CHUNK_R), :],
                                idx_v.at[ci % 2], sins[ci % 2])

    in_handles = [None, None]
    out_handles = [None, None]
    in_handles[0] = start_in(0)

    for ci in range(_NCHUNKS):
        slot = ci % 2
        in_handles[slot].wait()
        if ci + 1 < _NCHUNKS:
            in_handles[(ci + 1) % 2] = start_in(ci + 1)
        if out_handles[slot] is not None:
            out_handles[slot].wait()

        def row_body(r, c, slot=slot):
            for co in _COL_OFFS:
                ids = idx_v[slot, r, pl.ds(co, _LANES)]
                ids = jnp.minimum(jnp.maximum(ids, 0), 34)
                out_v[slot, r, pl.ds(co, _LANES)] = plsc.load_gather(
                    tbl_v, [ids])
            return c

        lax.fori_loop(0, _CHUNK_R, row_body, 0)
        out_handles[slot] = pltpu.async_copy(
            out_v.at[slot],
            out_hbm.at[pl.ds(base + ci * _CHUNK_R, _CHUNK_R), :], souts[slot])

    for h in out_handles:
        if h is not None:
            h.wait()


def kernel(discrete_f0, table):
    tbl = jnp.pad(table.reshape(-1), (0, _TBL_PAD - table.shape[0]))
    return _lut_gather(discrete_f0, tbl)


# speedup vs baseline: 1.4895x; 1.4895x over previous
import jax
import jax.numpy as jnp
from jax.experimental import pallas as pl

_B, _L = 16384, 200
_TC_BLOCK_R = 1024


def _tc_body(idx_ref, tbl_ref, out_ref):
    idx = jnp.minimum(jnp.maximum(idx_ref[...], 0), 34)
    src = jnp.broadcast_to(tbl_ref[...], (_TC_BLOCK_R, 128))
    out_ref[...] = jnp.take_along_axis(src, idx, axis=1,
                                       mode="promise_in_bounds")


_tc_gather = pl.pallas_call(
    _tc_body,
    grid=(_B // _TC_BLOCK_R,),
    in_specs=[
        pl.BlockSpec((_TC_BLOCK_R, _L), lambda i: (i, 0)),
        pl.BlockSpec((1, 128), lambda i: (0, 0)),
    ],
    out_specs=pl.BlockSpec((_TC_BLOCK_R, _L), lambda i: (i, 0)),
    out_shape=jax.ShapeDtypeStruct((_B, _L), jnp.float32),
)


def kernel(discrete_f0, table):
    idx = discrete_f0.astype(jnp.int32)
    centers = table.reshape(-1).astype(jnp.float32)
    tbl_tc = jnp.take(centers, jnp.minimum(jnp.arange(128), 34))[None, :]
    return _tc_gather(idx, tbl_tc)


# speedup vs baseline: 1.6117x; 1.0820x over previous
import jax
import jax.numpy as jnp
from jax.experimental import pallas as pl

_B, _L = 16384, 200
_TC_BLOCK_R = 2048


def _tc_body(idx_ref, tbl_ref, out_ref):
    idx = jnp.minimum(jnp.maximum(idx_ref[...], 0), 34)
    src = jnp.broadcast_to(tbl_ref[...], (_TC_BLOCK_R, 128))
    out_ref[...] = jnp.take_along_axis(src, idx, axis=1,
                                       mode="promise_in_bounds")


_tc_gather = pl.pallas_call(
    _tc_body,
    grid=(_B // _TC_BLOCK_R,),
    in_specs=[
        pl.BlockSpec((_TC_BLOCK_R, _L), lambda i: (i, 0)),
        pl.BlockSpec((1, 128), lambda i: (0, 0)),
    ],
    out_specs=pl.BlockSpec((_TC_BLOCK_R, _L), lambda i: (i, 0)),
    out_shape=jax.ShapeDtypeStruct((_B, _L), jnp.float32),
)


def kernel(discrete_f0, table):
    idx = discrete_f0.astype(jnp.int32)
    centers = table.reshape(-1).astype(jnp.float32)
    tbl_tc = jnp.take(centers, jnp.minimum(jnp.arange(128), 34))[None, :]
    return _tc_gather(idx, tbl_tc)


# speedup vs baseline: 1.6507x; 1.0242x over previous
import jax
import jax.numpy as jnp
from jax.experimental import pallas as pl

_B, _L = 16384, 200
_TC_BLOCK_R = 4096


def _tc_body(idx_ref, tbl_ref, out_ref):
    idx = jnp.minimum(jnp.maximum(idx_ref[...], 0), 34)
    src = jnp.broadcast_to(tbl_ref[...], (_TC_BLOCK_R, 128))
    out_ref[...] = jnp.take_along_axis(src, idx, axis=1,
                                       mode="promise_in_bounds")


_tc_gather = pl.pallas_call(
    _tc_body,
    grid=(_B // _TC_BLOCK_R,),
    in_specs=[
        pl.BlockSpec((_TC_BLOCK_R, _L), lambda i: (i, 0)),
        pl.BlockSpec((1, 128), lambda i: (0, 0)),
    ],
    out_specs=pl.BlockSpec((_TC_BLOCK_R, _L), lambda i: (i, 0)),
    out_shape=jax.ShapeDtypeStruct((_B, _L), jnp.float32),
)


def kernel(discrete_f0, table):
    idx = discrete_f0.astype(jnp.int32)
    centers = table.reshape(-1).astype(jnp.float32)
    tbl_tc = jnp.take(centers, jnp.minimum(jnp.arange(128), 34))[None, :]
    return _tc_gather(idx, tbl_tc)
